# baseline (speedup 1.0000x reference)
import functools

import jax
import jax.numpy as jnp
from jax import lax
from jax.experimental import pallas as pl
from jax.experimental.pallas import tpu as pltpu
from jax.experimental.pallas import tpu_sc as plsc

N_USERS = 5001
N_REAL = 10002
D = 128
E = 160000
HALF = 5120
NPAD = 2 * HALF
IOFF = HALF
DUMPP = 5056
NT = 16
EPT = E // NT
CH = 64
NFULL = EPT // CH
TAIL = EPT - NFULL * CH
NE2 = 162
NG = NE2 // 6
RPT = HALF // NT
HC = 64
NHC = RPT // HC
ZB = 32
HR = 48

_mesh = plsc.VectorSubcoreMesh(
    core_axis_name="c", subcore_axis_name="s", num_cores=2, num_subcores=16
)

_params = pltpu.CompilerParams(
    needs_layout_passes=False, use_tc_tiling_on_sc=False)


def _f32(x):
    return jnp.float32(x)


def _zero_zb(zb):
    def _zz(r, car):
        for k in range(8):
            zb[r, pl.ds(16 * k, 16)] = jnp.zeros((16,), jnp.float32)
        return car
    lax.fori_loop(0, ZB, _zz, 0)


def _rsqrt_vec(deg):
    m = deg > _f32(0.5)
    x = jnp.maximum(deg, _f32(1))
    i = lax.bitcast_convert_type(x, jnp.int32)
    y = lax.bitcast_convert_type(
        jnp.int32(0x5F3759DF) - lax.shift_right_logical(i, 1), jnp.float32)
    for _ in range(3):
        y = y * (_f32(1.5) - _f32(0.5) * x * y * y)
    return jnp.where(m, y, _f32(0)), jnp.where(m, _f32(1) / x, _f32(0))


@functools.partial(
    pl.kernel,
    out_type=(
        jax.ShapeDtypeStruct((3 * NPAD, D), jnp.float32),
        jax.ShapeDtypeStruct((3 * NPAD,), jnp.float32),
        jax.ShapeDtypeStruct((3 * NPAD,), jnp.float32),
        jax.ShapeDtypeStruct((2 * NT * 3 * NE2, CH), jnp.int32),
        jax.ShapeDtypeStruct((2 * NT * 3 * NE2, CH), jnp.int32),
    ),
    mesh=_mesh,
    compiler_params=_params,
    scratch_types=[
        pltpu.VMEM_SHARED((HR, D), jnp.float32),
        pltpu.VMEM((EPT + 80,), jnp.int32),
        pltpu.VMEM((EPT + 80,), jnp.int32),
        pltpu.VMEM((NE2, CH), jnp.int32),
        pltpu.VMEM((NE2, CH), jnp.int32),
        pltpu.VMEM((HC, D), jnp.float32),
        pltpu.VMEM((ZB, D), jnp.float32),
        pltpu.VMEM((RPT,), jnp.float32),
        pltpu.VMEM((RPT,), jnp.float32),
        pltpu.VMEM((HR,), jnp.int32),
    ],
)
def _setup_kernel(emb, eg0, eg1, eg2, g0_hbm, dinv_hbm, dinv2_hbm,
                  gi_hbm, si_hbm,
                  deg_s, u1d, o1d, g2d, s2d, buf, zb, dinv, dinv2, idx48):
    c = lax.axis_index("c")
    t = lax.axis_index("s")
    _zero_zb(zb)
    ones = jnp.ones((16,), jnp.float32)
    zv = jnp.zeros((16,), jnp.float32)

    def _iq(q, car):
        idx48[pl.ds(16 * q, 16)] = lax.iota(jnp.int32, 16) + 16 * q
        return car
    lax.fori_loop(0, HR // 16, _iq, 0)

    for b, eg in enumerate((eg0, eg1, eg2)):
        def _hz(r, car):
            for k in range(8):
                buf[r, pl.ds(16 * k, 16)] = zv
            return car
        lax.fori_loop(0, HR, _hz, 0)
        pltpu.sync_copy(zb.at[pl.ds(0, 3)], deg_s.at[pl.ds(3 * t, 3)])
        plsc.subcore_barrier()

        pltpu.sync_copy(eg.at[pl.ds(c * E + t * EPT, EPT)],
                        u1d.at[pl.ds(0, EPT)])
        pltpu.sync_copy(eg.at[pl.ds((1 - c) * E + t * EPT, EPT)],
                        o1d.at[pl.ds(0, EPT)])

        goff = (1 - c) * IOFF + b * NPAD

        def _fmt(j, car):
            for k in range(CH // 16):
                o = CH * j + 16 * k
                s2d[j, pl.ds(16 * k, 16)] = u1d[pl.ds(o, 16)]
                g2d[j, pl.ds(16 * k, 16)] = o1d[pl.ds(o, 16)] + goff
            return car
        lax.fori_loop(0, NFULL, _fmt, 0)
        s2d[NFULL, pl.ds(0, 16)] = u1d[pl.ds(NFULL * CH, 16)]
        g2d[NFULL, pl.ds(0, 16)] = o1d[pl.ds(NFULL * CH, 16)] + goff
        for k in range(TAIL // 16, (NE2 - NFULL) * (CH // 16)):
            jj, kk = NFULL + k // (CH // 16), k % (CH // 16)
            s2d[jj, pl.ds(16 * kk, 16)] = jnp.full((16,), DUMPP, jnp.int32)
            g2d[jj, pl.ds(16 * kk, 16)] = jnp.full(
                (16,), DUMPP, jnp.int32) + goff
        ibase = ((c * NT + t) * 3 + b) * NE2
        pltpu.sync_copy(g2d, gi_hbm.at[pl.ds(ibase, NE2)])
        pltpu.sync_copy(s2d, si_hbm.at[pl.ds(ibase, NE2)])

        def _hist(q, car):
            idx = u1d[pl.ds(16 * q, 16)]
            r = lax.shift_right_logical(idx, 7)
            cc = lax.bitwise_and(idx, 127)
            plsc.addupdate_scatter(buf, [r, cc], ones)
            return car
        lax.fori_loop(0, EPT // 16, _hist, 0)
        pltpu.sync_copy(buf.at[pl.ds(0, HR)], deg_s.at[idx48], add=True)
        plsc.subcore_barrier()

        pltpu.sync_copy(deg_s, buf.at[pl.ds(0, HR)])

        def _dv(q, car):
            fl = t * RPT + 16 * q
            rq = lax.shift_right_logical(fl, 7)
            cq = lax.bitwise_and(fl, 127)
            y, y2 = _rsqrt_vec(buf[rq, pl.ds(cq, 16)])
            dinv[pl.ds(16 * q, 16)] = y
            dinv2[pl.ds(16 * q, 16)] = y2
            return car
        lax.fori_loop(0, RPT // 16, _dv, 0)
        dbase = b * NPAD + c * HALF + t * RPT
        pltpu.sync_copy(dinv, dinv_hbm.at[pl.ds(dbase, RPT)])
        pltpu.sync_copy(dinv2, dinv2_hbm.at[pl.ds(dbase, RPT)])

        def _g0(jc, car):
            g = c * HALF + t * RPT + jc * HC
            pltpu.sync_copy(emb.at[pl.ds(g, HC)], buf.at[pl.ds(0, HC)])

            def _row(r, car2):
                sp = plsc.load_gather(
                    dinv, [jnp.full((16,), jc * HC + r, jnp.int32)])
                for k in range(8):
                    buf[r, pl.ds(16 * k, 16)] = buf[r, pl.ds(16 * k, 16)] * sp
                return car2
            lax.fori_loop(0, HC, _row, 0)
            pltpu.sync_copy(buf.at[pl.ds(0, HC)],
                            g0_hbm.at[pl.ds(b * NPAD + g, HC)])
            return car
        lax.fori_loop(0, NHC, _g0, 0)
        plsc.subcore_barrier()


def _layer_body(l, emb, gi_hbm, si_hbm, cg_in, hs_in, dv_hbm,
                cg_out, hs_out,
                acc, u2d, i2d, buf, zb, dloc,
                gs0, gs1, gs2, gs3, gs4, gs5, ss0, ss1, ss2, ss3, ss4, ss5):
    gsems = (gs0, gs1, gs2, gs3, gs4, gs5)
    ssems = (ss0, ss1, ss2, ss3, ss4, ss5)
    c = lax.axis_index("c")
    t = lax.axis_index("s")
    bufs = tuple(buf.at[pl.ds(CH * q, CH)] for q in range(6))
    _zero_zb(zb)

    def _drain(sem):
        pltpu.make_async_copy(cg_in.at[pl.ds(0, CH)], bufs[0], sem).wait()

    def _za(z, car):
        pltpu.sync_copy(zb, acc.at[pl.ds(t * RPT + ZB * z, ZB)])
        return car
    lax.fori_loop(0, RPT // ZB, _za, 0)
    plsc.subcore_barrier()

    for b in range(3):
        ibase = ((c * NT + t) * 3 + b) * NE2
        pltpu.sync_copy(gi_hbm.at[pl.ds(ibase, NE2)], u2d)
        pltpu.sync_copy(si_hbm.at[pl.ds(ibase, NE2)], i2d)

        def _edge_dir(gidx, sidx):
            pltpu.async_copy(cg_in.at[gidx.at[0]], bufs[0], gsems[0])
            pltpu.async_copy(cg_in.at[gidx.at[1]], bufs[1], gsems[1])

            def _grp(g, car):
                for q in range(6):
                    j = 6 * g + q
                    qn = (q + 2) % 6

                    @pl.when(j >= 4)
                    def _():
                        _drain(ssems[qn])

                    @pl.when(j < NE2 - 2)
                    def _():
                        pltpu.async_copy(cg_in.at[gidx.at[j + 2]],
                                         bufs[qn], gsems[qn])
                    _drain(gsems[q])
                    pltpu.async_copy(bufs[q], acc.at[sidx.at[j]], ssems[q],
                                     add=True)
                return car
            lax.fori_loop(0, NG, _grp, 0)
            for q in range(NE2 - 4, NE2):
                _drain(ssems[q % 6])

        _edge_dir(u2d, i2d)
        plsc.subcore_barrier()

        dbase = b * NPAD + c * HALF + t * RPT
        pltpu.sync_copy(dv_hbm.at[pl.ds(dbase, RPT)], dloc)

        def _epass(sc, car):
            lr = t * RPT + sc * HC
            fl = b * NPAD + c * HALF + lr
            pltpu.sync_copy(acc.at[pl.ds(lr, HC)], buf.at[pl.ds(0, HC)])
            if l == 1:
                pltpu.sync_copy(buf.at[pl.ds(0, HC)], hs_out.at[pl.ds(fl, HC)])

                def _row(r, car2):
                    sp = plsc.load_gather(
                        dloc, [jnp.full((16,), sc * HC + r, jnp.int32)])
                    for k in range(8):
                        buf[r, pl.ds(16 * k, 16)] = (
                            buf[r, pl.ds(16 * k, 16)] * sp)
                    return car2
                lax.fori_loop(0, HC, _row, 0)
                pltpu.sync_copy(buf.at[pl.ds(0, HC)], cg_out.at[pl.ds(fl, HC)])
            elif l == 2:
                pltpu.sync_copy(hs_in.at[pl.ds(fl, HC)], buf.at[pl.ds(HC, HC)])

                def _row(r, car2):
                    sp = plsc.load_gather(
                        dloc, [jnp.full((16,), sc * HC + r, jnp.int32)])
                    for k in range(8):
                        v = buf[r, pl.ds(16 * k, 16)]
                        buf[HC + r, pl.ds(16 * k, 16)] = (
                            buf[HC + r, pl.ds(16 * k, 16)] + v)
                        buf[r, pl.ds(16 * k, 16)] = v * sp
                    return car2
                lax.fori_loop(0, HC, _row, 0)
                pltpu.sync_copy(buf.at[pl.ds(0, HC)], cg_out.at[pl.ds(fl, HC)])
                pltpu.sync_copy(buf.at[pl.ds(HC, HC)], hs_out.at[pl.ds(fl, HC)])
            else:
                pltpu.sync_copy(hs_in.at[pl.ds(fl, HC)], buf.at[pl.ds(HC, HC)])
                pltpu.sync_copy(emb.at[pl.ds(fl - b * NPAD, HC)],
                                buf.at[pl.ds(2 * HC, HC)])

                def _row(r, car2):
                    sp = plsc.load_gather(
                        dloc, [jnp.full((16,), sc * HC + r, jnp.int32)])
                    for k in range(8):
                        hv = (buf[r, pl.ds(16 * k, 16)]
                              + buf[HC + r, pl.ds(16 * k, 16)])
                        buf[r, pl.ds(16 * k, 16)] = (
                            buf[2 * HC + r, pl.ds(16 * k, 16)]
                            + sp * hv) * _f32(0.25)
                    return car2
                lax.fori_loop(0, HC, _row, 0)
                pltpu.sync_copy(buf.at[pl.ds(0, HC)], hs_out.at[pl.ds(fl, HC)])
            for z in range(HC // ZB):
                pltpu.sync_copy(zb, acc.at[pl.ds(lr + ZB * z, ZB)])
            return car
        lax.fori_loop(0, NHC, _epass, 0)
        plsc.subcore_barrier()


def _make_layer(l):
    return functools.partial(
        pl.kernel,
        out_type=(
            jax.ShapeDtypeStruct((3 * NPAD, D), jnp.float32),
            jax.ShapeDtypeStruct((3 * NPAD, D), jnp.float32),
        ),
        mesh=_mesh,
        compiler_params=_params,
        scratch_types=[
            pltpu.VMEM_SHARED((HALF, D), jnp.float32),
            pltpu.VMEM((NE2, CH), jnp.int32),
            pltpu.VMEM((NE2, CH), jnp.int32),
            pltpu.VMEM((6 * CH, D), jnp.float32),
            pltpu.VMEM((ZB, D), jnp.float32),
            pltpu.VMEM((RPT,), jnp.float32),
            pltpu.SemaphoreType.DMA, pltpu.SemaphoreType.DMA,
            pltpu.SemaphoreType.DMA, pltpu.SemaphoreType.DMA,
            pltpu.SemaphoreType.DMA, pltpu.SemaphoreType.DMA,
            pltpu.SemaphoreType.DMA, pltpu.SemaphoreType.DMA,
            pltpu.SemaphoreType.DMA, pltpu.SemaphoreType.DMA,
            pltpu.SemaphoreType.DMA, pltpu.SemaphoreType.DMA,
        ],
    )(functools.partial(_layer_body, l))


_layer1 = _make_layer(1)
_layer2 = _make_layer(2)
_layer3 = _make_layer(3)


def kernel(embeddings, edge_index_0, edge_index_1, edge_index_2):
    emb_pad = (jnp.zeros((NPAD, D), jnp.float32)
               .at[:N_USERS].set(embeddings[:N_USERS])
               .at[IOFF:IOFF + N_USERS].set(embeddings[N_USERS:]))
    egs = [e.reshape(-1) for e in
           (edge_index_0, edge_index_1, edge_index_2)]
    g0, dinv, dinv2, gi, si = _setup_kernel(emb_pad, *egs)
    cg1, hs1 = _layer1(emb_pad, gi, si, g0, g0, dinv2)
    cg2, hs2 = _layer2(emb_pad, gi, si, cg1, hs1, dinv2)
    _, fin = _layer3(emb_pad, gi, si, cg2, hs2, dinv)
    outs = []
    for b in range(3):
        f = fin[b * NPAD:(b + 1) * NPAD]
        outs.append(jnp.concatenate([f[:N_USERS], f[IOFF:IOFF + N_USERS]]))
    return tuple(outs)


# speedup vs baseline: 1.4056x; 1.4056x over previous
import functools

import jax
import jax.numpy as jnp
from jax import lax
from jax.experimental import pallas as pl
from jax.experimental.pallas import tpu as pltpu
from jax.experimental.pallas import tpu_sc as plsc

N_USERS = 5001
N_REAL = 10002
D = 128
E = 160000
HALF = 5120
NPAD = 2 * HALF
IOFF = HALF
DUMPP = 5056
NT = 16
EPT = E // NT
CH = 32
NFULL = EPT // CH
TAIL = EPT - NFULL * CH
NE2 = 318
NG = NE2 // 6
RPT = HALF // NT
HC = 64
NHC = RPT // HC
ZB = 32
HR = 48

_mesh = plsc.VectorSubcoreMesh(
    core_axis_name="c", subcore_axis_name="s", num_cores=2, num_subcores=16
)

_params = pltpu.CompilerParams(
    needs_layout_passes=False, use_tc_tiling_on_sc=False)


def _f32(x):
    return jnp.float32(x)


def _zero_zb(zb):
    def _zz(r, car):
        for k in range(8):
            zb[r, pl.ds(16 * k, 16)] = jnp.zeros((16,), jnp.float32)
        return car
    lax.fori_loop(0, ZB, _zz, 0)


def _rsqrt_vec(deg):
    m = deg > _f32(0.5)
    x = jnp.maximum(deg, _f32(1))
    i = lax.bitcast_convert_type(x, jnp.int32)
    y = lax.bitcast_convert_type(
        jnp.int32(0x5F3759DF) - lax.shift_right_logical(i, 1), jnp.float32)
    for _ in range(3):
        y = y * (_f32(1.5) - _f32(0.5) * x * y * y)
    return jnp.where(m, y, _f32(0)), jnp.where(m, _f32(1) / x, _f32(0))


@functools.partial(
    pl.kernel,
    out_type=(
        jax.ShapeDtypeStruct((3 * NPAD, D), jnp.float32),
        jax.ShapeDtypeStruct((3 * NPAD,), jnp.float32),
        jax.ShapeDtypeStruct((3 * NPAD,), jnp.float32),
        jax.ShapeDtypeStruct((2 * NT * 3 * NE2, CH), jnp.int32),
        jax.ShapeDtypeStruct((2 * NT * 3 * NE2, CH), jnp.int32),
    ),
    mesh=_mesh,
    compiler_params=_params,
    scratch_types=[
        pltpu.VMEM_SHARED((HR, D), jnp.float32),
        pltpu.VMEM((EPT + 80,), jnp.int32),
        pltpu.VMEM((EPT + 80,), jnp.int32),
        pltpu.VMEM((NE2, CH), jnp.int32),
        pltpu.VMEM((NE2, CH), jnp.int32),
        pltpu.VMEM((HC, D), jnp.float32),
        pltpu.VMEM((ZB, D), jnp.float32),
        pltpu.VMEM((RPT,), jnp.float32),
        pltpu.VMEM((RPT,), jnp.float32),
        pltpu.VMEM((HR,), jnp.int32),
    ],
)
def _setup_kernel(emb, eg0, eg1, eg2, g0_hbm, dinv_hbm, dinv2_hbm,
                  gi_hbm, si_hbm,
                  deg_s, u1d, o1d, g2d, s2d, buf, zb, dinv, dinv2, idx48):
    c = lax.axis_index("c")
    t = lax.axis_index("s")
    _zero_zb(zb)
    ones = jnp.ones((16,), jnp.float32)
    zv = jnp.zeros((16,), jnp.float32)

    def _iq(q, car):
        idx48[pl.ds(16 * q, 16)] = lax.iota(jnp.int32, 16) + 16 * q
        return car
    lax.fori_loop(0, HR // 16, _iq, 0)

    for b, eg in enumerate((eg0, eg1, eg2)):
        def _hz(r, car):
            for k in range(8):
                buf[r, pl.ds(16 * k, 16)] = zv
            return car
        lax.fori_loop(0, HR, _hz, 0)
        pltpu.sync_copy(zb.at[pl.ds(0, 3)], deg_s.at[pl.ds(3 * t, 3)])
        plsc.subcore_barrier()

        pltpu.sync_copy(eg.at[pl.ds(c * E + t * EPT, EPT)],
                        u1d.at[pl.ds(0, EPT)])
        pltpu.sync_copy(eg.at[pl.ds((1 - c) * E + t * EPT, EPT)],
                        o1d.at[pl.ds(0, EPT)])

        goff = (1 - c) * IOFF + b * NPAD

        def _fmt(j, car):
            for k in range(CH // 16):
                o = CH * j + 16 * k
                s2d[j, pl.ds(16 * k, 16)] = u1d[pl.ds(o, 16)]
                g2d[j, pl.ds(16 * k, 16)] = o1d[pl.ds(o, 16)] + goff
            return car
        lax.fori_loop(0, NFULL, _fmt, 0)
        s2d[NFULL, pl.ds(0, 16)] = u1d[pl.ds(NFULL * CH, 16)]
        g2d[NFULL, pl.ds(0, 16)] = o1d[pl.ds(NFULL * CH, 16)] + goff
        for k in range(TAIL // 16, (NE2 - NFULL) * (CH // 16)):
            jj, kk = NFULL + k // (CH // 16), k % (CH // 16)
            s2d[jj, pl.ds(16 * kk, 16)] = jnp.full((16,), DUMPP, jnp.int32)
            g2d[jj, pl.ds(16 * kk, 16)] = jnp.full(
                (16,), DUMPP, jnp.int32) + goff
        ibase = ((c * NT + t) * 3 + b) * NE2
        pltpu.sync_copy(g2d, gi_hbm.at[pl.ds(ibase, NE2)])
        pltpu.sync_copy(s2d, si_hbm.at[pl.ds(ibase, NE2)])

        def _hist(q, car):
            idx = u1d[pl.ds(16 * q, 16)]
            r = lax.shift_right_logical(idx, 7)
            cc = lax.bitwise_and(idx, 127)
            plsc.addupdate_scatter(buf, [r, cc], ones)
            return car
        lax.fori_loop(0, EPT // 16, _hist, 0)
        pltpu.sync_copy(buf.at[pl.ds(0, HR)], deg_s.at[idx48], add=True)
        plsc.subcore_barrier()

        pltpu.sync_copy(deg_s, buf.at[pl.ds(0, HR)])

        def _dv(q, car):
            fl = t * RPT + 16 * q
            rq = lax.shift_right_logical(fl, 7)
            cq = lax.bitwise_and(fl, 127)
            y, y2 = _rsqrt_vec(buf[rq, pl.ds(cq, 16)])
            dinv[pl.ds(16 * q, 16)] = y
            dinv2[pl.ds(16 * q, 16)] = y2
            return car
        lax.fori_loop(0, RPT // 16, _dv, 0)
        dbase = b * NPAD + c * HALF + t * RPT
        pltpu.sync_copy(dinv, dinv_hbm.at[pl.ds(dbase, RPT)])
        pltpu.sync_copy(dinv2, dinv2_hbm.at[pl.ds(dbase, RPT)])

        def _g0(jc, car):
            g = c * HALF + t * RPT + jc * HC
            pltpu.sync_copy(emb.at[pl.ds(g, HC)], buf.at[pl.ds(0, HC)])

            def _row(r, car2):
                sp = plsc.load_gather(
                    dinv, [jnp.full((16,), jc * HC + r, jnp.int32)])
                for k in range(8):
                    buf[r, pl.ds(16 * k, 16)] = buf[r, pl.ds(16 * k, 16)] * sp
                return car2
            lax.fori_loop(0, HC, _row, 0)
            pltpu.sync_copy(buf.at[pl.ds(0, HC)],
                            g0_hbm.at[pl.ds(b * NPAD + g, HC)])
            return car
        lax.fori_loop(0, NHC, _g0, 0)
        plsc.subcore_barrier()


def _layer_body(l, emb, gi_hbm, si_hbm, cg_in, hs_in, dv_hbm,
                cg_out, hs_out,
                acc, u2d, i2d, buf, zb, dloc,
                gs0, gs1, gs2, gs3, gs4, gs5, ss0, ss1, ss2, ss3, ss4, ss5):
    gsems = (gs0, gs1, gs2, gs3, gs4, gs5)
    ssems = (ss0, ss1, ss2, ss3, ss4, ss5)
    c = lax.axis_index("c")
    t = lax.axis_index("s")
    bufs = tuple(buf.at[pl.ds(CH * q, CH)] for q in range(6))
    _zero_zb(zb)

    def _drain(sem):
        pltpu.make_async_copy(cg_in.at[pl.ds(0, CH)], bufs[0], sem).wait()

    def _za(z, car):
        pltpu.sync_copy(zb, acc.at[pl.ds(t * RPT + ZB * z, ZB)])
        return car
    lax.fori_loop(0, RPT // ZB, _za, 0)
    plsc.subcore_barrier()

    for b in range(3):
        ibase = ((c * NT + t) * 3 + b) * NE2
        pltpu.sync_copy(gi_hbm.at[pl.ds(ibase, NE2)], u2d)
        pltpu.sync_copy(si_hbm.at[pl.ds(ibase, NE2)], i2d)

        def _edge_dir(gidx, sidx):
            pltpu.async_copy(cg_in.at[gidx.at[0]], bufs[0], gsems[0])
            pltpu.async_copy(cg_in.at[gidx.at[1]], bufs[1], gsems[1])

            def _grp(g, car):
                for q in range(6):
                    j = 6 * g + q
                    qn = (q + 2) % 6

                    @pl.when(j >= 4)
                    def _():
                        _drain(ssems[qn])

                    @pl.when(j < NE2 - 2)
                    def _():
                        pltpu.async_copy(cg_in.at[gidx.at[j + 2]],
                                         bufs[qn], gsems[qn])
                    _drain(gsems[q])
                    pltpu.async_copy(bufs[q], acc.at[sidx.at[j]], ssems[q],
                                     add=True)
                return car
            lax.fori_loop(0, NG, _grp, 0)
            for q in range(NE2 - 4, NE2):
                _drain(ssems[q % 6])

        _edge_dir(u2d, i2d)
        plsc.subcore_barrier()

        dbase = b * NPAD + c * HALF + t * RPT
        pltpu.sync_copy(dv_hbm.at[pl.ds(dbase, RPT)], dloc)

        def _epass(sc, car):
            lr = t * RPT + sc * HC
            fl = b * NPAD + c * HALF + lr
            pltpu.sync_copy(acc.at[pl.ds(lr, HC)], buf.at[pl.ds(0, HC)])
            if l == 1:
                pltpu.sync_copy(buf.at[pl.ds(0, HC)], hs_out.at[pl.ds(fl, HC)])

                def _row(r, car2):
                    sp = plsc.load_gather(
                        dloc, [jnp.full((16,), sc * HC + r, jnp.int32)])
                    for k in range(8):
                        buf[r, pl.ds(16 * k, 16)] = (
                            buf[r, pl.ds(16 * k, 16)] * sp)
                    return car2
                lax.fori_loop(0, HC, _row, 0)
                pltpu.sync_copy(buf.at[pl.ds(0, HC)], cg_out.at[pl.ds(fl, HC)])
            elif l == 2:
                pltpu.sync_copy(hs_in.at[pl.ds(fl, HC)], buf.at[pl.ds(HC, HC)])

                def _row(r, car2):
                    sp = plsc.load_gather(
                        dloc, [jnp.full((16,), sc * HC + r, jnp.int32)])
                    for k in range(8):
                        v = buf[r, pl.ds(16 * k, 16)]
                        buf[HC + r, pl.ds(16 * k, 16)] = (
                            buf[HC + r, pl.ds(16 * k, 16)] + v)
                        buf[r, pl.ds(16 * k, 16)] = v * sp
                    return car2
                lax.fori_loop(0, HC, _row, 0)
                pltpu.sync_copy(buf.at[pl.ds(0, HC)], cg_out.at[pl.ds(fl, HC)])
                pltpu.sync_copy(buf.at[pl.ds(HC, HC)], hs_out.at[pl.ds(fl, HC)])
            else:
                pltpu.sync_copy(hs_in.at[pl.ds(fl, HC)], buf.at[pl.ds(HC, HC)])
                pltpu.sync_copy(emb.at[pl.ds(fl - b * NPAD, HC)],
                                buf.at[pl.ds(2 * HC, HC)])

                def _row(r, car2):
                    sp = plsc.load_gather(
                        dloc, [jnp.full((16,), sc * HC + r, jnp.int32)])
                    for k in range(8):
                        hv = (buf[r, pl.ds(16 * k, 16)]
                              + buf[HC + r, pl.ds(16 * k, 16)])
                        buf[r, pl.ds(16 * k, 16)] = (
                            buf[2 * HC + r, pl.ds(16 * k, 16)]
                            + sp * hv) * _f32(0.25)
                    return car2
                lax.fori_loop(0, HC, _row, 0)
                pltpu.sync_copy(buf.at[pl.ds(0, HC)], hs_out.at[pl.ds(fl, HC)])
            for z in range(HC // ZB):
                pltpu.sync_copy(zb, acc.at[pl.ds(lr + ZB * z, ZB)])
            return car
        lax.fori_loop(0, NHC, _epass, 0)
        plsc.subcore_barrier()


def _make_layer(l):
    return functools.partial(
        pl.kernel,
        out_type=(
            jax.ShapeDtypeStruct((3 * NPAD, D), jnp.float32),
            jax.ShapeDtypeStruct((3 * NPAD, D), jnp.float32),
        ),
        mesh=_mesh,
        compiler_params=_params,
        scratch_types=[
            pltpu.VMEM_SHARED((HALF, D), jnp.float32),
            pltpu.VMEM((NE2, CH), jnp.int32),
            pltpu.VMEM((NE2, CH), jnp.int32),
            pltpu.VMEM((6 * CH, D), jnp.float32),
            pltpu.VMEM((ZB, D), jnp.float32),
            pltpu.VMEM((RPT,), jnp.float32),
            pltpu.SemaphoreType.DMA, pltpu.SemaphoreType.DMA,
            pltpu.SemaphoreType.DMA, pltpu.SemaphoreType.DMA,
            pltpu.SemaphoreType.DMA, pltpu.SemaphoreType.DMA,
            pltpu.SemaphoreType.DMA, pltpu.SemaphoreType.DMA,
            pltpu.SemaphoreType.DMA, pltpu.SemaphoreType.DMA,
            pltpu.SemaphoreType.DMA, pltpu.SemaphoreType.DMA,
        ],
    )(functools.partial(_layer_body, l))


_layer1 = _make_layer(1)
_layer2 = _make_layer(2)
_layer3 = _make_layer(3)


def kernel(embeddings, edge_index_0, edge_index_1, edge_index_2):
    emb_pad = (jnp.zeros((NPAD, D), jnp.float32)
               .at[:N_USERS].set(embeddings[:N_USERS])
               .at[IOFF:IOFF + N_USERS].set(embeddings[N_USERS:]))
    egs = [e.reshape(-1) for e in
           (edge_index_0, edge_index_1, edge_index_2)]
    g0, dinv, dinv2, gi, si = _setup_kernel(emb_pad, *egs)
    cg1, hs1 = _layer1(emb_pad, gi, si, g0, g0, dinv2)
    cg2, hs2 = _layer2(emb_pad, gi, si, cg1, hs1, dinv2)
    _, fin = _layer3(emb_pad, gi, si, cg2, hs2, dinv)
    outs = []
    for b in range(3):
        f = fin[b * NPAD:(b + 1) * NPAD]
        outs.append(jnp.concatenate([f[:N_USERS], f[IOFF:IOFF + N_USERS]]))
    return tuple(outs)
